# baseline (speedup 1.0000x reference)
import functools
import math

import jax
import jax.numpy as jnp
from jax import lax
from jax.experimental import pallas as pl
from jax.experimental.pallas import tpu as pltpu
from jax.experimental.pallas import tpu_sc as plsc

N_NODES = 256
D = 128
ROWS = 64
G_SC = 2
NC = 2
NS = 16
NW = NC * NS
RPW = G_SC * N_NODES // NW
DC = D // 16


def _attn_weights_kernel(x_ref, wq_ref, wk_ref, w_ref, m_ref):
    b = pl.program_id(0)

    @pl.when(b == 0)
    def _():
        m = jnp.dot(wq_ref[:].T, wk_ref[:], preferred_element_type=jnp.float32)
        m_ref[:] = m * (1.0 / math.sqrt(D))

    xm = jnp.dot(x_ref[0], m_ref[:], preferred_element_type=jnp.float32)
    logits = jnp.dot(xm, x_ref[0].T, preferred_element_type=jnp.float32)
    e = jnp.exp(logits - jnp.max(logits, axis=-1, keepdims=True))
    w_ref[0] = e / jnp.sum(e, axis=-1, keepdims=True)


def _tc_stream_kernel(x_ref, wq_ref, wk_ref, adj_ref, out_ref, m_ref):
    b = pl.program_id(0)
    ib = pl.program_id(1)

    @pl.when(jnp.logical_and(b == 0, ib == 0))
    def _():
        m = jnp.dot(wq_ref[:].T, wk_ref[:], preferred_element_type=jnp.float32)
        m_ref[:] = m * (1.0 / math.sqrt(D))

    x_rows = x_ref[0, pl.ds(ib * ROWS, ROWS), :]
    xm = jnp.dot(x_rows, m_ref[:], preferred_element_type=jnp.float32)
    logits = jnp.dot(xm, x_ref[0].T, preferred_element_type=jnp.float32)
    e = jnp.exp(logits - jnp.max(logits, axis=-1, keepdims=True))
    acc = jax.lax.dot_general(
        e, adj_ref[0],
        dimension_numbers=(((1,), (1,)), ((0,), (0,))),
        preferred_element_type=jnp.float32,
    )
    out_ref[0] = acc / jnp.sum(e, axis=-1, keepdims=True)


def _sc_reduce_kernel(adj_hbm, w_hbm, out_hbm, row_buf, w_buf, out_buf):
    wid = lax.axis_index("s") * NC + lax.axis_index("c")
    base = wid * RPW

    def row_body(r, carry):
        row = base + r
        b = row // N_NODES
        i = row - b * N_NODES
        pltpu.sync_copy(adj_hbm.at[b, i], row_buf)
        pltpu.sync_copy(w_hbm.at[b, i], w_buf)

        def j_chunk(jc, accs):
            wv = w_buf[pl.ds(jc * 16, 16)]
            for jj in range(16):
                wj = wv[jj]
                j = jc * 16 + jj
                accs = tuple(
                    accs[dc] + wj * row_buf[j, pl.ds(dc * 16, 16)]
                    for dc in range(DC)
                )
            return accs

        accs = lax.fori_loop(
            0, N_NODES // 16, j_chunk,
            tuple(jnp.zeros((16,), jnp.float32) for _ in range(DC)),
        )
        for dc in range(DC):
            out_buf[pl.ds(dc * 16, 16)] = accs[dc]
        pltpu.sync_copy(out_buf, out_hbm.at[b, i])
        return carry

    lax.fori_loop(0, RPW, row_body, 0)


@jax.jit
def kernel(x, adj_matrix, W_q, W_k):
    B = adj_matrix.shape[0]
    xg = x.reshape(B, N_NODES, D)

    w_sc = pl.pallas_call(
        _attn_weights_kernel,
        grid=(G_SC,),
        in_specs=[
            pl.BlockSpec((1, N_NODES, D), lambda b: (b, 0, 0)),
            pl.BlockSpec((D, D), lambda b: (0, 0)),
            pl.BlockSpec((D, D), lambda b: (0, 0)),
        ],
        out_specs=pl.BlockSpec((1, N_NODES, N_NODES), lambda b: (b, 0, 0)),
        out_shape=jax.ShapeDtypeStruct((G_SC, N_NODES, N_NODES), jnp.float32),
        scratch_shapes=[pltpu.VMEM((D, D), jnp.float32)],
    )(xg[:G_SC], W_q, W_k)

    sc_fn = functools.partial(
        pl.kernel,
        mesh=plsc.VectorSubcoreMesh(core_axis_name="c", subcore_axis_name="s"),
        out_type=jax.ShapeDtypeStruct((G_SC, N_NODES, D), jnp.float32),
        scratch_types=[
            pltpu.VMEM((N_NODES, D), jnp.float32),
            pltpu.VMEM((N_NODES,), jnp.float32),
            pltpu.VMEM((D,), jnp.float32),
        ],
    )(_sc_reduce_kernel)
    out_sc = sc_fn(adj_matrix, w_sc)

    out_tc = pl.pallas_call(
        _tc_stream_kernel,
        grid=(B - G_SC, N_NODES // ROWS),
        in_specs=[
            pl.BlockSpec((1, N_NODES, D), lambda b, i: (b + G_SC, 0, 0)),
            pl.BlockSpec((D, D), lambda b, i: (0, 0)),
            pl.BlockSpec((D, D), lambda b, i: (0, 0)),
            pl.BlockSpec((1, ROWS, N_NODES, D), lambda b, i: (b + G_SC, i, 0, 0)),
        ],
        out_specs=pl.BlockSpec((1, ROWS, D), lambda b, i: (b, i, 0)),
        out_shape=jax.ShapeDtypeStruct((B - G_SC, N_NODES, D), jnp.float32),
        scratch_shapes=[pltpu.VMEM((D, D), jnp.float32)],
    )(xg, W_q, W_k, adj_matrix)

    return jnp.concatenate([out_sc, out_tc], axis=0)


# speedup vs baseline: 1.0563x; 1.0563x over previous
import functools
import math

import jax
import jax.numpy as jnp
from jax import lax
from jax.experimental import pallas as pl
from jax.experimental.pallas import tpu as pltpu
from jax.experimental.pallas import tpu_sc as plsc

N_NODES = 256
D = 128
ROWS = 64
G_SC = 2
NC = 2
NS = 16
NW = NC * NS
RPW = G_SC * N_NODES // NW
DC = D // 16


def _attn_weights_kernel(x_ref, wq_ref, wk_ref, w_ref, m_ref):
    b = pl.program_id(0)

    @pl.when(b == 0)
    def _():
        m = jnp.dot(wq_ref[:].T, wk_ref[:], preferred_element_type=jnp.float32)
        m_ref[:] = m * (1.0 / math.sqrt(D))

    xm = jnp.dot(x_ref[0], m_ref[:], preferred_element_type=jnp.float32)
    logits = jnp.dot(xm, x_ref[0].T, preferred_element_type=jnp.float32)
    e = jnp.exp(logits - jnp.max(logits, axis=-1, keepdims=True))
    w_ref[0] = e / jnp.sum(e, axis=-1, keepdims=True)


def _tc_stream_kernel(x_ref, wq_ref, wk_ref, adj_ref, out_ref, m_ref):
    b = pl.program_id(0)
    ib = pl.program_id(1)

    @pl.when(jnp.logical_and(b == 0, ib == 0))
    def _():
        m = jnp.dot(wq_ref[:].T, wk_ref[:], preferred_element_type=jnp.float32)
        m_ref[:] = m * (1.0 / math.sqrt(D))

    x_rows = x_ref[0, pl.ds(ib * ROWS, ROWS), :]
    xm = jnp.dot(x_rows, m_ref[:], preferred_element_type=jnp.float32)
    logits = jnp.dot(xm, x_ref[0].T, preferred_element_type=jnp.float32)
    e = jnp.exp(logits - jnp.max(logits, axis=-1, keepdims=True))
    acc = jax.lax.dot_general(
        e, adj_ref[0],
        dimension_numbers=(((1,), (1,)), ((0,), (0,))),
        preferred_element_type=jnp.float32,
    )
    out_ref[0] = acc / jnp.sum(e, axis=-1, keepdims=True)


def _sc_reduce_kernel(adj_hbm, w_hbm, out_hbm,
                      buf0, buf1, w_all, out_all, sem0, sem1):
    wid = lax.axis_index("s") * NC + lax.axis_index("c")
    base = wid * RPW

    pltpu.sync_copy(w_hbm.at[pl.ds(base, RPW)], w_all)
    pltpu.async_copy(adj_hbm.at[base], buf0, sem0)
    pltpu.async_copy(adj_hbm.at[base + 1], buf1, sem1)

    def compute_row(r, buf):
        def j_chunk(jc, accs):
            wv = w_all[r, pl.ds(jc * 16, 16)]
            for jj in range(16):
                wj = wv[jj]
                j = jc * 16 + jj
                accs = tuple(
                    accs[dc] + wj * buf[j, pl.ds(dc * 16, 16)]
                    for dc in range(DC)
                )
            return accs

        accs = lax.fori_loop(
            0, N_NODES // 16, j_chunk,
            tuple(jnp.zeros((16,), jnp.float32) for _ in range(DC)),
        )
        for dc in range(DC):
            out_all[r, pl.ds(dc * 16, 16)] = accs[dc]

    def row_pair(r0, carry):
        for k, (buf, sem) in enumerate(((buf0, sem0), (buf1, sem1))):
            r = r0 + k
            pltpu.make_async_copy(adj_hbm.at[base], buf, sem).wait()
            compute_row(r, buf)

            @pl.when(r + 2 < RPW)
            def _():
                pltpu.async_copy(adj_hbm.at[base + r + 2], buf, sem)
        return carry

    lax.fori_loop(0, RPW // 2, lambda t, c: row_pair(t * 2, c), 0)
    pltpu.sync_copy(out_all, out_hbm.at[pl.ds(base, RPW)])


@jax.jit
def kernel(x, adj_matrix, W_q, W_k):
    B = adj_matrix.shape[0]
    xg = x.reshape(B, N_NODES, D)

    w_sc = pl.pallas_call(
        _attn_weights_kernel,
        grid=(G_SC,),
        in_specs=[
            pl.BlockSpec((1, N_NODES, D), lambda b: (b, 0, 0)),
            pl.BlockSpec((D, D), lambda b: (0, 0)),
            pl.BlockSpec((D, D), lambda b: (0, 0)),
        ],
        out_specs=pl.BlockSpec((1, N_NODES, N_NODES), lambda b: (b, 0, 0)),
        out_shape=jax.ShapeDtypeStruct((G_SC, N_NODES, N_NODES), jnp.float32),
        scratch_shapes=[pltpu.VMEM((D, D), jnp.float32)],
    )(xg[:G_SC], W_q, W_k)

    sc_fn = functools.partial(
        pl.kernel,
        mesh=plsc.VectorSubcoreMesh(core_axis_name="c", subcore_axis_name="s"),
        out_type=jax.ShapeDtypeStruct((G_SC * N_NODES, D), jnp.float32),
        scratch_types=[
            pltpu.VMEM((N_NODES, D), jnp.float32),
            pltpu.VMEM((N_NODES, D), jnp.float32),
            pltpu.VMEM((RPW, N_NODES), jnp.float32),
            pltpu.VMEM((RPW, D), jnp.float32),
            pltpu.SemaphoreType.DMA,
            pltpu.SemaphoreType.DMA,
        ],
    )(_sc_reduce_kernel)
    out_sc = sc_fn(
        adj_matrix.reshape(B * N_NODES, N_NODES, D),
        w_sc.reshape(G_SC * N_NODES, N_NODES),
    ).reshape(G_SC, N_NODES, D)

    out_tc = pl.pallas_call(
        _tc_stream_kernel,
        grid=(B - G_SC, N_NODES // ROWS),
        in_specs=[
            pl.BlockSpec((1, N_NODES, D), lambda b, i: (b + G_SC, 0, 0)),
            pl.BlockSpec((D, D), lambda b, i: (0, 0)),
            pl.BlockSpec((D, D), lambda b, i: (0, 0)),
            pl.BlockSpec((1, ROWS, N_NODES, D), lambda b, i: (b + G_SC, i, 0, 0)),
        ],
        out_specs=pl.BlockSpec((1, ROWS, D), lambda b, i: (b, i, 0)),
        out_shape=jax.ShapeDtypeStruct((B - G_SC, N_NODES, D), jnp.float32),
        scratch_shapes=[pltpu.VMEM((D, D), jnp.float32)],
    )(xg, W_q, W_k, adj_matrix)

    return jnp.concatenate([out_sc, out_tc], axis=0)


# speedup vs baseline: 1.3701x; 1.2971x over previous
import math

import jax
import jax.numpy as jnp
from jax.experimental import pallas as pl
from jax.experimental.pallas import tpu as pltpu

N_NODES = 256
D = 128
ROWS = 64


def _edge_attn_kernel(x_ref, wq_ref, wk_ref, adj_ref, out_ref, m_ref):
    b = pl.program_id(0)
    ib = pl.program_id(1)

    @pl.when(jnp.logical_and(b == 0, ib == 0))
    def _():
        m = jnp.dot(wq_ref[:].T, wk_ref[:], preferred_element_type=jnp.float32)
        m_ref[:] = m * (1.0 / math.sqrt(D))

    x_rows = x_ref[0, pl.ds(ib * ROWS, ROWS), :]
    xm = jnp.dot(x_rows, m_ref[:], preferred_element_type=jnp.float32)
    logits = jnp.dot(xm, x_ref[0].T, preferred_element_type=jnp.float32)
    e = jnp.exp(logits - jnp.max(logits, axis=-1, keepdims=True))
    acc = jax.lax.dot_general(
        e, adj_ref[0],
        dimension_numbers=(((1,), (1,)), ((0,), (0,))),
        preferred_element_type=jnp.float32,
    )
    out_ref[0] = acc / jnp.sum(e, axis=-1, keepdims=True)


@jax.jit
def kernel(x, adj_matrix, W_q, W_k):
    B = adj_matrix.shape[0]
    xg = x.reshape(B, N_NODES, D)
    grid = (B, N_NODES // ROWS)
    out = pl.pallas_call(
        _edge_attn_kernel,
        grid=grid,
        in_specs=[
            pl.BlockSpec((1, N_NODES, D), lambda b, i: (b, 0, 0)),
            pl.BlockSpec((D, D), lambda b, i: (0, 0)),
            pl.BlockSpec((D, D), lambda b, i: (0, 0)),
            pl.BlockSpec((1, ROWS, N_NODES, D), lambda b, i: (b, i, 0, 0)),
        ],
        out_specs=pl.BlockSpec((1, ROWS, D), lambda b, i: (b, i, 0)),
        out_shape=jax.ShapeDtypeStruct((B, N_NODES, D), jnp.float32),
        scratch_shapes=[pltpu.VMEM((D, D), jnp.float32)],
    )(xg, W_q, W_k, adj_matrix)
    return out
